# baseline (speedup 1.0000x reference)
import functools

import jax
import jax.numpy as jnp
from jax import lax
from jax.experimental import pallas as pl
from jax.experimental.pallas import tpu as pltpu, tpu_sc as plsc

NC, NS = 2, 16
NW = NC * NS
CHR = 16
UNROLL = 8


def _sc_pe_add(x2, pe, B):
    R, D = x2.shape
    S = pe.shape[0]
    rpw = S // NW
    npc = rpw // CHR
    G = npc * B
    mesh = plsc.VectorSubcoreMesh(core_axis_name="c", subcore_axis_name="s")

    @functools.partial(
        pl.kernel,
        out_type=jax.ShapeDtypeStruct((R, D), x2.dtype),
        mesh=mesh,
        compiler_params=pltpu.CompilerParams(use_tc_tiling_on_sc=True),
        scratch_types=[
            [pltpu.VMEM((CHR, D), jnp.float32) for _ in range(4)],
            [pltpu.VMEM((CHR, D), jnp.float32) for _ in range(2)],
            [pltpu.SemaphoreType.DMA for _ in range(4)],
            [pltpu.SemaphoreType.DMA for _ in range(4)],
            [pltpu.SemaphoreType.DMA for _ in range(2)],
        ],
    )
    def k(x_hbm, pe_hbm, out_hbm, xb, pb, slx, ssx, sp):
        wid = lax.axis_index("s") * NC + lax.axis_index("c")
        per0 = wid * rpw

        def x_row(g):
            p, b = divmod(g, B)
            return b * S + per0 + p * CHR

        def compute(xbuf, pbuf):
            @pl.loop(0, CHR)
            def _(r):
                @pl.loop(0, D, step=16, unroll=UNROLL)
                def _(c):
                    plsc.addupdate(xbuf.at[r, pl.ds(c, 16)],
                                   pbuf[r, pl.ds(c, 16)])

        peloads = [None] * npc
        xloads = [None] * G
        xstores = [None] * G
        peloads[0] = pltpu.async_copy(
            pe_hbm.at[pl.ds(per0, CHR)], pb[0], sp[0])
        for g in range(min(4, G)):
            xloads[g] = pltpu.async_copy(
                x_hbm.at[pl.ds(x_row(g), CHR)], xb[g % 4], slx[g % 4])
        for g in range(G):
            p, b = divmod(g, B)
            bi = g % 4
            if b == 0:
                if p + 1 < npc:
                    peloads[p + 1] = pltpu.async_copy(
                        pe_hbm.at[pl.ds(per0 + (p + 1) * CHR, CHR)],
                        pb[(p + 1) % 2], sp[(p + 1) % 2])
                peloads[p].wait()
            if g >= 3 and g + 1 < G:
                xstores[g - 3].wait()
                xloads[g + 1] = pltpu.async_copy(
                    x_hbm.at[pl.ds(x_row(g + 1), CHR)],
                    xb[(g + 1) % 4], slx[(g + 1) % 4])
            xloads[g].wait()
            compute(xb[bi], pb[p % 2])
            xstores[g] = pltpu.async_copy(
                xb[bi], out_hbm.at[pl.ds(x_row(g), CHR)], ssx[bi])
        for g in range(max(0, G - 3), G):
            xstores[g].wait()

    return k(x2, pe)


def kernel(x, pe_table):
    B, S, D = x.shape
    out = _sc_pe_add(x.reshape(B * S, D), pe_table[:S], B)
    return out.reshape(B, S, D)


# speedup vs baseline: 1.8227x; 1.8227x over previous
import functools

import jax
import jax.numpy as jnp
from jax import lax
from jax.experimental import pallas as pl
from jax.experimental.pallas import tpu as pltpu, tpu_sc as plsc

NC, NS = 2, 16
NW = NC * NS
CHR = 16
UNROLL = 32


def _sc_pe_add(x2, pe, B):
    R, D = x2.shape
    S = pe.shape[0]
    rpw = S // NW
    npc = rpw // CHR
    G = npc * B
    mesh = plsc.VectorSubcoreMesh(core_axis_name="c", subcore_axis_name="s")

    @functools.partial(
        pl.kernel,
        out_type=jax.ShapeDtypeStruct((R, D), x2.dtype),
        mesh=mesh,
        compiler_params=pltpu.CompilerParams(use_tc_tiling_on_sc=True),
        scratch_types=[
            [pltpu.VMEM((CHR, D), jnp.float32) for _ in range(4)],
            [pltpu.VMEM((CHR, D), jnp.float32) for _ in range(2)],
            [pltpu.SemaphoreType.DMA for _ in range(4)],
            [pltpu.SemaphoreType.DMA for _ in range(4)],
            [pltpu.SemaphoreType.DMA for _ in range(2)],
        ],
    )
    def k(x_hbm, pe_hbm, out_hbm, xb, pb, slx, ssx, sp):
        wid = lax.axis_index("s") * NC + lax.axis_index("c")
        per0 = wid * rpw

        def x_row(g):
            p, b = divmod(g, B)
            return b * S + per0 + p * CHR

        def compute(xbuf, pbuf):
            @pl.loop(0, CHR)
            def _(r):
                @pl.loop(0, D, step=16, unroll=UNROLL)
                def _(c):
                    plsc.addupdate(xbuf.at[r, pl.ds(c, 16)],
                                   pbuf[r, pl.ds(c, 16)])

        peloads = [None] * npc
        xloads = [None] * G
        xstores = [None] * G
        peloads[0] = pltpu.async_copy(
            pe_hbm.at[pl.ds(per0, CHR)], pb[0], sp[0])
        for g in range(min(4, G)):
            xloads[g] = pltpu.async_copy(
                x_hbm.at[pl.ds(x_row(g), CHR)], xb[g % 4], slx[g % 4])
        for g in range(G):
            p, b = divmod(g, B)
            bi = g % 4
            if b == 0:
                if p + 1 < npc:
                    peloads[p + 1] = pltpu.async_copy(
                        pe_hbm.at[pl.ds(per0 + (p + 1) * CHR, CHR)],
                        pb[(p + 1) % 2], sp[(p + 1) % 2])
                peloads[p].wait()
            if g >= 3 and g + 1 < G:
                xstores[g - 3].wait()
                xloads[g + 1] = pltpu.async_copy(
                    x_hbm.at[pl.ds(x_row(g + 1), CHR)],
                    xb[(g + 1) % 4], slx[(g + 1) % 4])
            xloads[g].wait()
            compute(xb[bi], pb[p % 2])
            xstores[g] = pltpu.async_copy(
                xb[bi], out_hbm.at[pl.ds(x_row(g), CHR)], ssx[bi])
        for g in range(max(0, G - 3), G):
            xstores[g].wait()

    return k(x2, pe)


def kernel(x, pe_table):
    B, S, D = x.shape
    out = _sc_pe_add(x.reshape(B * S, D), pe_table[:S], B)
    return out.reshape(B, S, D)
